# baseline (speedup 1.0000x reference)
import functools

import jax
import jax.numpy as jnp
from jax import lax
from jax.experimental import pallas as pl
from jax.experimental.pallas import tpu as pltpu
from jax.experimental.pallas import tpu_sc as plsc

_E = 64
_K = 6
_NC = 2
_NS = 16
_NW = _NC * _NS
_L = 16


def _probs_kernel(x_ref, w_ref, p_ref):
    x = x_ref[...]
    w = w_ref[...]
    logits = lax.dot_general(w, x, (((1,), (1,)), ((), ())),
                             preferred_element_type=jnp.float32)
    m = jnp.max(logits, axis=0, keepdims=True)
    e = jnp.exp(logits - m)
    p_ref[...] = e / jnp.sum(e, axis=0, keepdims=True)


def _make_probs(nc, d, t, chunk):
    blk0 = chunk * (nc // t)
    return pl.pallas_call(
        _probs_kernel,
        grid=(nc // t,),
        in_specs=[
            pl.BlockSpec((t, d), lambda i: (blk0 + i, 0)),
            pl.BlockSpec((_E, d), lambda i: (0, 0)),
        ],
        out_specs=pl.BlockSpec((_E, t), lambda i: (0, i)),
        out_shape=jax.ShapeDtypeStruct((_E, nc), jnp.float32),
    )


def _insert(v, ie, tt, ii):
    c = [v > t for t in tt]
    nt = [jnp.where(c[0], v, tt[0])]
    ni = [jnp.where(c[0], ie, ii[0])]
    for j in range(1, _K):
        nt.append(jnp.where(c[j - 1], tt[j - 1], jnp.where(c[j], v, tt[j])))
        ni.append(jnp.where(c[j - 1], ii[j - 1], jnp.where(c[j], ie, ii[j])))
    return nt, ni


def _make_sc_topk(n):
    tpw = n // _NW
    ng = tpw // _L
    mesh = plsc.VectorSubcoreMesh(core_axis_name="c", subcore_axis_name="s")

    @functools.partial(
        pl.kernel,
        out_type=[
            jax.ShapeDtypeStruct((_K, n), jnp.float32),
            jax.ShapeDtypeStruct((_K, n), jnp.int32),
        ],
        mesh=mesh,
        scratch_types=[
            pltpu.VMEM((_E, tpw), jnp.float32),
            pltpu.VMEM((_K, tpw), jnp.float32),
            pltpu.VMEM((_K, tpw), jnp.int32),
        ],
    )
    def sc_topk(p_hbm, wout_hbm, iout_hbm, p_v, w_v, i_v):
        cid = lax.axis_index("c")
        sid = lax.axis_index("s")
        wid = sid * _NC + cid
        base_tok = wid * tpw
        pltpu.sync_copy(p_hbm.at[:, pl.ds(base_tok, tpw)], p_v)
        lane = lax.broadcasted_iota(jnp.int32, (_L,), 0)
        zi = jnp.zeros((_L,), jnp.int32)
        neg = jnp.full((_L,), -1.0, jnp.float32)

        def group_body(g, carry):
            base = g * _L

            def e_body(e, tc_):
                tt, ii = tc_[:_K], tc_[_K:]
                v = p_v[e, pl.ds(base, _L)]
                nt, ni = _insert(v, zi + e, tt, ii)
                return tuple(nt) + tuple(ni)

            res = lax.fori_loop(0, _E, e_body, (neg,) * _K + (zi,) * _K)
            for k in range(_K):
                w_v[k, pl.ds(base, _L)] = res[k]
                i_v[k, pl.ds(base, _L)] = res[_K + k]
            return carry

        lax.fori_loop(0, ng, group_body, 0)
        pltpu.sync_copy(w_v, wout_hbm.at[:, pl.ds(base_tok, tpw)])
        pltpu.sync_copy(i_v, iout_hbm.at[:, pl.ds(base_tok, tpw)])

    return sc_topk


def kernel(x, weight):
    n, d = x.shape
    n_chunks = 1
    nc = n // n_chunks
    sc_topk = _make_sc_topk(nc)
    wts, its = [], []
    for c in range(n_chunks):
        probs = _make_probs(nc, d, 1024, c)(x, weight)
        w_t, i_t = sc_topk(probs)
        wts.append(w_t)
        its.append(i_t)
    return (jnp.concatenate(wts, axis=1).T,
            jnp.concatenate(its, axis=1).T)


# speedup vs baseline: 1.0472x; 1.0472x over previous
import functools

import jax
import jax.numpy as jnp
from jax import lax
from jax.experimental import pallas as pl
from jax.experimental.pallas import tpu as pltpu
from jax.experimental.pallas import tpu_sc as plsc

_E = 64
_K = 6
_NC = 2
_NS = 16
_NW = _NC * _NS
_L = 16
_UNROLL = 4


def _probs_kernel(x_ref, w_ref, p_ref):
    x = x_ref[...]
    w = w_ref[...]
    logits = lax.dot_general(w, x, (((1,), (1,)), ((), ())),
                             preferred_element_type=jnp.float32)
    m = jnp.max(logits, axis=0, keepdims=True)
    e = jnp.exp(logits - m)
    p_ref[...] = e / jnp.sum(e, axis=0, keepdims=True)


def _make_probs(nc, d, t, chunk):
    blk0 = chunk * (nc // t)
    return pl.pallas_call(
        _probs_kernel,
        grid=(nc // t,),
        in_specs=[
            pl.BlockSpec((t, d), lambda i: (blk0 + i, 0)),
            pl.BlockSpec((_E, d), lambda i: (0, 0)),
        ],
        out_specs=pl.BlockSpec((_E, t), lambda i: (0, i)),
        out_shape=jax.ShapeDtypeStruct((_E, nc), jnp.float32),
    )


def _insert(v, ie, tt, ii):
    c = [v > t for t in tt]
    nt = [jnp.where(c[0], v, tt[0])]
    ni = [jnp.where(c[0], ie, ii[0])]
    for j in range(1, _K):
        nt.append(jnp.where(c[j - 1], tt[j - 1], jnp.where(c[j], v, tt[j])))
        ni.append(jnp.where(c[j - 1], ii[j - 1], jnp.where(c[j], ie, ii[j])))
    return nt, ni


def _make_sc_topk(n):
    tpw = n // _NW
    ng = tpw // _L
    mesh = plsc.VectorSubcoreMesh(core_axis_name="c", subcore_axis_name="s")

    @functools.partial(
        pl.kernel,
        out_type=[
            jax.ShapeDtypeStruct((_K, n), jnp.float32),
            jax.ShapeDtypeStruct((_K, n), jnp.int32),
        ],
        mesh=mesh,
        scratch_types=[
            pltpu.VMEM((_E, tpw), jnp.float32),
            pltpu.VMEM((_K, tpw), jnp.float32),
            pltpu.VMEM((_K, tpw), jnp.int32),
        ],
    )
    def sc_topk(p_hbm, wout_hbm, iout_hbm, p_v, w_v, i_v):
        cid = lax.axis_index("c")
        sid = lax.axis_index("s")
        wid = sid * _NC + cid
        base_tok = wid * tpw
        pltpu.sync_copy(p_hbm.at[:, pl.ds(base_tok, tpw)], p_v)
        lane = lax.broadcasted_iota(jnp.int32, (_L,), 0)
        zi = jnp.zeros((_L,), jnp.int32)
        neg = jnp.full((_L,), -1.0, jnp.float32)

        def group_body(g, carry):
            base = g * _L

            def e_body(eo, tc_):
                tt, ii = list(tc_[:_K]), list(tc_[_K:])
                for u in range(_UNROLL):
                    e = eo * _UNROLL + u
                    v = p_v[e, pl.ds(base, _L)]
                    tt, ii = _insert(v, zi + e, tt, ii)
                return tuple(tt) + tuple(ii)

            res = lax.fori_loop(0, _E // _UNROLL, e_body,
                                (neg,) * _K + (zi,) * _K)
            for k in range(_K):
                w_v[k, pl.ds(base, _L)] = res[k]
                i_v[k, pl.ds(base, _L)] = res[_K + k]
            return carry

        lax.fori_loop(0, ng, group_body, 0)
        pltpu.sync_copy(w_v, wout_hbm.at[:, pl.ds(base_tok, tpw)])
        pltpu.sync_copy(i_v, iout_hbm.at[:, pl.ds(base_tok, tpw)])

    return sc_topk


def kernel(x, weight):
    n, d = x.shape
    n_chunks = 4
    nc = n // n_chunks
    sc_topk = _make_sc_topk(nc)
    wts, its = [], []
    for c in range(n_chunks):
        probs = _make_probs(nc, d, 1024, c)(x, weight)
        w_t, i_t = sc_topk(probs)
        wts.append(w_t)
        its.append(i_t)
    return (jnp.concatenate(wts, axis=1).T,
            jnp.concatenate(its, axis=1).T)
